# baseline (speedup 1.0000x reference)
import functools

import jax
import jax.numpy as jnp
from jax import lax
from jax.experimental import pallas as pl
from jax.experimental.pallas import tpu as pltpu
from jax.experimental.pallas import tpu_sc as plsc

N_NODE = 10000
E_EDGE = 320000
D_FEAT = 128

NC = 2
NS = 16
EPT = E_EDGE // NS
CH = 64
NCHUNK = EPT // CH
TCH = EPT - NCHUNK * CH
NSLOT = 6

ROWS_PT = 624
TAIL_OFF = NS * ROWS_PT
TAIL = N_NODE - TAIL_OFF


def _sc_body(xu, xi, su, du, si, di, zeros, out_u_pre, out_i_pre, *scratch):
    c = lax.axis_index("c")
    s = lax.axis_index("s")
    it = iter(scratch)
    idx_s = tuple(next(it) for _ in range(NSLOT))
    idx_d = tuple(next(it) for _ in range(NSLOT))
    idx_st = next(it)
    idx_dt = next(it)
    rows = tuple(next(it) for _ in range(NSLOT))
    acc = next(it)
    gsem = tuple(next(it) for _ in range(NSLOT))
    ssem = tuple(next(it) for _ in range(NSLOT))
    esem = tuple(next(it) for _ in range(NSLOT))
    dsem = tuple(next(it) for _ in range(NSLOT))
    zsem = next(it)
    tsem_s = next(it)
    tsem_d = next(it)

    pltpu.async_copy(zeros, acc.at[pl.ds(s * ROWS_PT, ROWS_PT)], zsem)

    @pl.when(s == NS - 1)
    def _():
        pltpu.sync_copy(zeros.at[pl.ds(0, TAIL)],
                        acc.at[pl.ds(TAIL_OFF, TAIL)])

    def make_ops(x_hbm, src_hbm, dst_hbm):
        base0 = s * EPT

        def start_src_idx(j, b):
            pltpu.async_copy(src_hbm.at[pl.ds(base0 + j * CH, CH)],
                             idx_s[b], esem[b])

        def start_dst_idx(j, b):
            pltpu.async_copy(dst_hbm.at[pl.ds(base0 + j * CH, CH)],
                             idx_d[b], dsem[b])

        def wait_src_idx(j, b):
            pltpu.make_async_copy(src_hbm.at[pl.ds(base0 + j * CH, CH)],
                                  idx_s[b], esem[b]).wait()

        def wait_dst_idx(j, b):
            pltpu.make_async_copy(dst_hbm.at[pl.ds(base0 + j * CH, CH)],
                                  idx_d[b], dsem[b]).wait()

        def start_gather(b):
            return pltpu.async_copy(x_hbm.at[idx_s[b]], rows[b], gsem[b])

        def wait_gather(b):
            pltpu.make_async_copy(x_hbm.at[idx_s[b]], rows[b],
                                  gsem[b]).wait()

        def start_scatter(b):
            return pltpu.async_copy(rows[b], acc.at[idx_d[b]], ssem[b],
                                    add=True)

        def wait_scatter(b):
            pltpu.make_async_copy(rows[b], acc.at[idx_d[b]], ssem[b]).wait()

        tbase = base0 + NCHUNK * CH

        def prologue():
            for k in range(NSLOT - 1):
                start_src_idx(k, k)
            for k in range(NSLOT - 3):
                start_dst_idx(k, k)
            pltpu.async_copy(src_hbm.at[pl.ds(tbase, TCH)], idx_st, tsem_s)
            pltpu.async_copy(dst_hbm.at[pl.ds(tbase, TCH)], idx_dt, tsem_d)
            for k in range(NSLOT - 3):
                wait_src_idx(k, k)
                start_gather(k)

        def mainloop():
            @pl.loop(0, NCHUNK, step=NSLOT)
            def _outer(i):
                for b in range(NSLOT):
                    j = i + b
                    n3 = (b + 3) % NSLOT
                    n5 = (b + 5) % NSLOT

                    if b <= 2:
                        @pl.when(i > 0)
                        def _():
                            wait_scatter(n3)
                    else:
                        wait_scatter(n3)

                    def _next_gather(j=j, n3=n3):
                        wait_src_idx(j + 3, n3)
                        start_gather(n3)

                    if b <= 2:
                        _next_gather()
                    else:
                        pl.when(i < NCHUNK - NSLOT)(_next_gather)

                    wait_gather(b)
                    wait_dst_idx(j, b)
                    start_scatter(b)

                    def _prefetch_src(j=j, n5=n5):
                        start_src_idx(j + 5, n5)

                    def _prefetch_dst(j=j, n3=n3):
                        start_dst_idx(j + 3, n3)

                    if b == 0:
                        _prefetch_src()
                    else:
                        pl.when(i < NCHUNK - NSLOT)(_prefetch_src)
                    if b <= 2:
                        _prefetch_dst()
                    else:
                        pl.when(i < NCHUNK - NSLOT)(_prefetch_dst)

            for k in range(3):
                wait_scatter((NCHUNK - 3 + k) % NSLOT)

            pltpu.make_async_copy(src_hbm.at[pl.ds(tbase, TCH)], idx_st,
                                  tsem_s).wait()
            pltpu.make_async_copy(dst_hbm.at[pl.ds(tbase, TCH)], idx_dt,
                                  tsem_d).wait()
            pltpu.async_copy(x_hbm.at[idx_st], rows[0].at[pl.ds(0, TCH)],
                             gsem[0]).wait()
            pltpu.sync_copy(rows[0].at[pl.ds(0, TCH)], acc.at[idx_dt],
                            add=True)

        return prologue, mainloop

    pro_u, main_u = make_ops(xu, su, du)
    pro_i, main_i = make_ops(xi, si, di)

    @pl.when(c == 0)
    def _():
        pro_u()

    @pl.when(c == 1)
    def _():
        pro_i()

    pltpu.make_async_copy(zeros, acc.at[pl.ds(s * ROWS_PT, ROWS_PT)],
                          zsem).wait()
    plsc.subcore_barrier()

    @pl.when(c == 0)
    def _():
        main_u()

    @pl.when(c == 1)
    def _():
        main_i()

    plsc.subcore_barrier()

    def dump(out_ref):
        pltpu.sync_copy(acc.at[pl.ds(s * ROWS_PT, ROWS_PT)],
                        out_ref.at[pl.ds(s * ROWS_PT, ROWS_PT)])

        @pl.when(s == NS - 1)
        def _():
            pltpu.sync_copy(acc.at[pl.ds(TAIL_OFF, TAIL)],
                            out_ref.at[pl.ds(TAIL_OFF, TAIL)])

    @pl.when(c == 0)
    def _():
        dump(out_i_pre)

    @pl.when(c == 1)
    def _():
        dump(out_u_pre)


_sc_segment_sum = pl.kernel(
    _sc_body,
    out_type=(
        jax.ShapeDtypeStruct((N_NODE, D_FEAT), jnp.float32),
        jax.ShapeDtypeStruct((N_NODE, D_FEAT), jnp.float32),
    ),
    mesh=plsc.VectorSubcoreMesh(
        core_axis_name="c", subcore_axis_name="s",
        num_cores=NC, num_subcores=NS),
    scratch_types=(
        [pltpu.VMEM((CH,), jnp.int32) for _ in range(NSLOT)] +
        [pltpu.VMEM((CH,), jnp.int32) for _ in range(NSLOT)] +
        [pltpu.VMEM((TCH,), jnp.int32) for _ in range(2)] +
        [pltpu.VMEM((CH, D_FEAT), jnp.float32) for _ in range(NSLOT)] +
        [pltpu.VMEM_SHARED((N_NODE, D_FEAT), jnp.float32)] +
        [pltpu.SemaphoreType.DMA for _ in range(4 * NSLOT + 3)]
    ),
)


def _mm_body(pu_ref, pi_ref, wu_ref, wi_ref, ou_ref, oi_ref):
    ou_ref[...] = jnp.dot(pu_ref[...], wi_ref[...],
                          preferred_element_type=jnp.float32)
    oi_ref[...] = jnp.dot(pi_ref[...], wu_ref[...],
                          preferred_element_type=jnp.float32)


_MM_BLK = 1000


def _apply_weights(p_user, p_item, W_u2i, W_i2u):
    grid = (N_NODE // _MM_BLK,)
    blk = pl.BlockSpec((_MM_BLK, D_FEAT), lambda i: (i, 0))
    wblk = pl.BlockSpec((D_FEAT, D_FEAT), lambda i: (0, 0))
    return pl.pallas_call(
        _mm_body,
        grid=grid,
        in_specs=[blk, blk, wblk, wblk],
        out_specs=[blk, blk],
        out_shape=(
            jax.ShapeDtypeStruct((N_NODE, D_FEAT), jnp.float32),
            jax.ShapeDtypeStruct((N_NODE, D_FEAT), jnp.float32),
        ),
    )(p_user, p_item, W_u2i, W_i2u)


@jax.jit
def kernel(x_user, x_item, W_u2i, W_i2u,
           src_idx_u2i, dst_idx_u2i, src_idx_i2u, dst_idx_i2u):
    zeros = jnp.zeros((ROWS_PT, D_FEAT), jnp.float32)
    p_user, p_item = _sc_segment_sum(
        x_user, x_item, src_idx_u2i, dst_idx_u2i, src_idx_i2u, dst_idx_i2u,
        zeros)
    out_user, out_item = _apply_weights(p_user, p_item, W_u2i, W_i2u)
    return (out_user, out_item)


# speedup vs baseline: 1.0151x; 1.0151x over previous
import functools

import jax
import jax.numpy as jnp
from jax import lax
from jax.experimental import pallas as pl
from jax.experimental.pallas import tpu as pltpu
from jax.experimental.pallas import tpu_sc as plsc

N_NODE = 10000
E_EDGE = 320000
D_FEAT = 128

NC = 2
NS = 16
EPT = E_EDGE // NS
CH = 64
NCHUNK = EPT // CH
TCH = EPT - NCHUNK * CH
NSLOT = 6

ROWS_PT = 624
TAIL_OFF = NS * ROWS_PT
TAIL = N_NODE - TAIL_OFF


def _sc_body(xu, xi, su, du, si, di, zeros, out_u_pre, out_i_pre, *scratch):
    c = lax.axis_index("c")
    s = lax.axis_index("s")
    it = iter(scratch)
    idx_s = tuple(next(it) for _ in range(NSLOT))
    idx_d = tuple(next(it) for _ in range(NSLOT))
    idx_st = next(it)
    idx_dt = next(it)
    rows = tuple(next(it) for _ in range(NSLOT))
    acc = next(it)
    gsem = tuple(next(it) for _ in range(NSLOT))
    ssem = tuple(next(it) for _ in range(NSLOT))
    esem = tuple(next(it) for _ in range(NSLOT))
    dsem = tuple(next(it) for _ in range(NSLOT))
    zsem = next(it)
    tsem_s = next(it)
    tsem_d = next(it)

    pltpu.async_copy(zeros, acc.at[pl.ds(s * ROWS_PT, ROWS_PT)], zsem)

    @pl.when(s == NS - 1)
    def _():
        pltpu.sync_copy(zeros.at[pl.ds(0, TAIL)],
                        acc.at[pl.ds(TAIL_OFF, TAIL)])

    def make_ops(x_hbm, src_hbm, dst_hbm):
        base0 = s * EPT

        def start_src_idx(j, b):
            pltpu.async_copy(src_hbm.at[pl.ds(base0 + j * CH, CH)],
                             idx_s[b], esem[b])

        def start_dst_idx(j, b):
            pltpu.async_copy(dst_hbm.at[pl.ds(base0 + j * CH, CH)],
                             idx_d[b], dsem[b])

        def wait_src_idx(j, b):
            pltpu.make_async_copy(src_hbm.at[pl.ds(base0 + j * CH, CH)],
                                  idx_s[b], esem[b]).wait()

        def wait_dst_idx(j, b):
            pltpu.make_async_copy(dst_hbm.at[pl.ds(base0 + j * CH, CH)],
                                  idx_d[b], dsem[b]).wait()

        def start_gather(b):
            return pltpu.async_copy(x_hbm.at[idx_s[b]], rows[b], gsem[b])

        def wait_gather(b):
            pltpu.make_async_copy(x_hbm.at[idx_s[b]], rows[b],
                                  gsem[b]).wait()

        def start_scatter(b):
            return pltpu.async_copy(rows[b], acc.at[idx_d[b]], ssem[b],
                                    add=True)

        def wait_scatter(b):
            pltpu.make_async_copy(rows[b], acc.at[idx_d[b]], ssem[b]).wait()

        tbase = base0 + NCHUNK * CH

        def prologue():
            for k in range(NSLOT - 1):
                start_src_idx(k, k)
            for k in range(NSLOT - 3):
                start_dst_idx(k, k)
            pltpu.async_copy(src_hbm.at[pl.ds(tbase, TCH)], idx_st, tsem_s)
            pltpu.async_copy(dst_hbm.at[pl.ds(tbase, TCH)], idx_dt, tsem_d)
            for k in range(NSLOT - 3):
                wait_src_idx(k, k)
                start_gather(k)

        def mainloop():
            @pl.loop(0, NCHUNK, step=NSLOT)
            def _outer(i):
                for b in range(NSLOT):
                    j = i + b
                    n3 = (b + 3) % NSLOT
                    n5 = (b + 5) % NSLOT

                    if b <= 2:
                        @pl.when(i > 0)
                        def _():
                            wait_scatter(n3)
                    else:
                        wait_scatter(n3)

                    def _next_gather(j=j, n3=n3):
                        wait_src_idx(j + 3, n3)
                        start_gather(n3)

                    if b <= 2:
                        _next_gather()
                    else:
                        pl.when(i < NCHUNK - NSLOT)(_next_gather)

                    wait_gather(b)
                    wait_dst_idx(j, b)
                    start_scatter(b)

                    def _prefetch_src(j=j, n5=n5):
                        start_src_idx(j + 5, n5)

                    def _prefetch_dst(j=j, n3=n3):
                        start_dst_idx(j + 3, n3)

                    if b == 0:
                        _prefetch_src()
                    else:
                        pl.when(i < NCHUNK - NSLOT)(_prefetch_src)
                    if b <= 2:
                        _prefetch_dst()
                    else:
                        pl.when(i < NCHUNK - NSLOT)(_prefetch_dst)

            for k in range(3):
                wait_scatter((NCHUNK - 3 + k) % NSLOT)

            pltpu.make_async_copy(src_hbm.at[pl.ds(tbase, TCH)], idx_st,
                                  tsem_s).wait()
            pltpu.make_async_copy(dst_hbm.at[pl.ds(tbase, TCH)], idx_dt,
                                  tsem_d).wait()
            pltpu.async_copy(x_hbm.at[idx_st], rows[0].at[pl.ds(0, TCH)],
                             gsem[0]).wait()
            pltpu.sync_copy(rows[0].at[pl.ds(0, TCH)], acc.at[idx_dt],
                            add=True)

        return prologue, mainloop

    pro_u, main_u = make_ops(xu, su, du)
    pro_i, main_i = make_ops(xi, si, di)

    @pl.when(c == 0)
    def _():
        pro_u()

    @pl.when(c == 1)
    def _():
        pro_i()

    pltpu.make_async_copy(zeros, acc.at[pl.ds(s * ROWS_PT, ROWS_PT)],
                          zsem).wait()
    plsc.subcore_barrier()

    @pl.when(c == 0)
    def _():
        main_u()

    @pl.when(c == 1)
    def _():
        main_i()

    plsc.subcore_barrier()

    def dump(out_ref):
        pltpu.sync_copy(acc.at[pl.ds(s * ROWS_PT, ROWS_PT)],
                        out_ref.at[pl.ds(s * ROWS_PT, ROWS_PT)])

        @pl.when(s == NS - 1)
        def _():
            pltpu.sync_copy(acc.at[pl.ds(TAIL_OFF, TAIL)],
                            out_ref.at[pl.ds(TAIL_OFF, TAIL)])

    @pl.when(c == 0)
    def _():
        dump(out_i_pre)

    @pl.when(c == 1)
    def _():
        dump(out_u_pre)


_sc_segment_sum = pl.kernel(
    _sc_body,
    out_type=(
        jax.ShapeDtypeStruct((N_NODE, D_FEAT), jnp.float32),
        jax.ShapeDtypeStruct((N_NODE, D_FEAT), jnp.float32),
    ),
    mesh=plsc.VectorSubcoreMesh(
        core_axis_name="c", subcore_axis_name="s",
        num_cores=NC, num_subcores=NS),
    scratch_types=(
        [pltpu.VMEM((CH,), jnp.int32) for _ in range(NSLOT)] +
        [pltpu.VMEM((CH,), jnp.int32) for _ in range(NSLOT)] +
        [pltpu.VMEM((TCH,), jnp.int32) for _ in range(2)] +
        [pltpu.VMEM((CH, D_FEAT), jnp.float32) for _ in range(NSLOT)] +
        [pltpu.VMEM_SHARED((N_NODE, D_FEAT), jnp.float32)] +
        [pltpu.SemaphoreType.DMA for _ in range(4 * NSLOT + 3)]
    ),
)


def _mm_body(pu_ref, pi_ref, wu_ref, wi_ref, ou_ref, oi_ref):
    ou_ref[...] = jnp.dot(pu_ref[...], wi_ref[...],
                          preferred_element_type=jnp.float32)
    oi_ref[...] = jnp.dot(pi_ref[...], wu_ref[...],
                          preferred_element_type=jnp.float32)


_MM_BLK = 2000


def _apply_weights(p_user, p_item, W_u2i, W_i2u):
    grid = (N_NODE // _MM_BLK,)
    blk = pl.BlockSpec((_MM_BLK, D_FEAT), lambda i: (i, 0))
    wblk = pl.BlockSpec((D_FEAT, D_FEAT), lambda i: (0, 0))
    return pl.pallas_call(
        _mm_body,
        grid=grid,
        in_specs=[blk, blk, wblk, wblk],
        out_specs=[blk, blk],
        out_shape=(
            jax.ShapeDtypeStruct((N_NODE, D_FEAT), jnp.float32),
            jax.ShapeDtypeStruct((N_NODE, D_FEAT), jnp.float32),
        ),
    )(p_user, p_item, W_u2i, W_i2u)


@jax.jit
def kernel(x_user, x_item, W_u2i, W_i2u,
           src_idx_u2i, dst_idx_u2i, src_idx_i2u, dst_idx_i2u):
    zeros = jnp.zeros((ROWS_PT, D_FEAT), jnp.float32)
    p_user, p_item = _sc_segment_sum(
        x_user, x_item, src_idx_u2i, dst_idx_u2i, src_idx_i2u, dst_idx_i2u,
        zeros)
    out_user, out_item = _apply_weights(p_user, p_item, W_u2i, W_i2u)
    return (out_user, out_item)
